# baseline (speedup 1.0000x reference)
import functools

import jax
import jax.numpy as jnp
from jax import lax
from jax.experimental import pallas as pl
from jax.experimental.pallas import tpu as pltpu
from jax.experimental.pallas import tpu_sc as plsc

_B = 4096
_L = 64
_D = 64
_V = 1000000
_NPAIR = _B * _L
_NW = 32
_PER_W = _NPAIR // _NW
_C = 128
_CHUNKS = _PER_W // _C
_VB = 16384


def _conv_body(pa_ref, pb_ref, eye_ref, out_ref):
    out_ref[:, :_D] = pa_ref[...].T
    out_ref[:, _D:] = pb_ref[...].T


@jax.jit
def _convert_pair(pa, pb, eye):
    grid = (pl.cdiv(_V, _VB),)
    in_spec = pl.BlockSpec((_D, _VB), lambda j: (0, j))
    return pl.pallas_call(
        _conv_body,
        grid=grid,
        in_specs=[in_spec, in_spec, pl.BlockSpec((_D, _D), lambda j: (0, 0))],
        out_specs=pl.BlockSpec((_VB, 2 * _D), lambda j: (j, 0)),
        out_shape=jax.ShapeDtypeStruct((_V, 2 * _D), jnp.float32),
    )(pa, pb, eye)


_NTILE = _V // 128
_CPW = (_NTILE + 1 + _NW - 1) // _NW


def _sc_conv_body(pa_hbm, pb_hbm, tail_hbm, out_hbm, bufs, gsem, wsem):
    c = lax.axis_index("c")
    s = lax.axis_index("s")
    wid = s * 2 + c

    didx = [lax.iota(jnp.int32, 16) + (g * 16) for g in range(4)]

    def fire_in(k, bset):
        col = pl.ds(k * 128, 128)
        pltpu.async_copy(pa_hbm.at[:, col], bset[0], gsem)
        pltpu.async_copy(pb_hbm.at[:, col], bset[1], gsem)

    def wait_in(k, bset):
        col = pl.ds(k * 128, 128)
        pltpu.make_async_copy(pa_hbm.at[:, col], bset[0], gsem).wait()
        pltpu.make_async_copy(pb_hbm.at[:, col], bset[1], gsem).wait()

    def transpose_into(bset):
        buf_a, buf_b, orow = bset

        def row(r, carry):
            vidx = jnp.full((16,), r, jnp.int32)
            for g in range(4):
                orow[r, pl.ds(g * 16, 16)] = plsc.load_gather(
                    buf_a, [didx[g], vidx])
                orow[r, pl.ds(_D + g * 16, 16)] = plsc.load_gather(
                    buf_b, [didx[g], vidx])
            return carry

        lax.fori_loop(0, 128, row, 0, unroll=2)

    def fire_out(k, bset):
        pltpu.async_copy(bset[2], out_hbm.at[pl.ds(k * 128, 128)], wsem)

    def wait_out(k, bset):
        pltpu.make_async_copy(bset[2], out_hbm.at[pl.ds(k * 128, 128)],
                              wsem).wait()

    k0 = wid

    @pl.when(k0 < _NTILE)
    def _():
        fire_in(k0, bufs[0])

    def outer(t, carry):
        for b in range(2):
            k = (t * 2 + b) * _NW + wid
            cur = bufs[b]
            other = bufs[1 - b]
            kn = k + _NW

            @pl.when(k < _NTILE)
            def _():
                @pl.when(k >= _NW)
                def _():
                    wait_out(k - _NW, other)

                @pl.when(kn < _NTILE)
                def _():
                    fire_in(kn, other)

                wait_in(k, cur)
                transpose_into(cur)
                fire_out(k, cur)
        return carry

    lax.fori_loop(0, (_CPW + 1) // 2, outer, 0)
    last_t = (_NTILE - 1 - wid) // _NW
    last = last_t * _NW + wid

    @pl.when(last_t % 2 == 0)
    def _():
        wait_out(last, bufs[0])

    @pl.when(last_t % 2 == 1)
    def _():
        wait_out(last, bufs[1])

    @pl.when(wid == 0)
    def _():
        buf_a = bufs[0][0]
        pltpu.sync_copy(tail_hbm, buf_a)
        pltpu.sync_copy(buf_a, out_hbm.at[pl.ds(_NTILE * 128, 64)])


def _tail_body(pa_ref, pb_ref, out_ref):
    out_ref[:, :_D] = pa_ref[...].T
    out_ref[:, _D:] = pb_ref[...].T


@jax.jit
def _sc_convert_pair(pa, pb):
    tail = pl.pallas_call(
        _tail_body,
        out_shape=jax.ShapeDtypeStruct((64, 2 * _D), jnp.float32),
    )(pa[:, _NTILE * 128:], pb[:, _NTILE * 128:])
    mesh = plsc.VectorSubcoreMesh(core_axis_name="c", subcore_axis_name="s")
    bset = (pltpu.VMEM((_D, 128), jnp.float32),
            pltpu.VMEM((_D, 128), jnp.float32),
            pltpu.VMEM((128, 2 * _D), jnp.float32))
    fn = pl.kernel(
        _sc_conv_body,
        out_type=jax.ShapeDtypeStruct((_V, 2 * _D), jnp.float32),
        mesh=mesh,
        compiler_params=pltpu.CompilerParams(needs_layout_passes=False),
        scratch_types=[
            (bset, bset),
            pltpu.SemaphoreType.DMA,
            pltpu.SemaphoreType.DMA,
        ],
    )
    return fn(pa, pb, tail)


def _sc_gather_body(idx_hbm, tab_hbm, out_hbm, idx_all, bufs, gsem, wsem):
    c = lax.axis_index("c")
    s = lax.axis_index("s")
    wid = s * 2 + c
    wbase = wid * _PER_W

    pltpu.sync_copy(idx_hbm.at[pl.ds(wbase, _PER_W)], idx_all)

    def idx_slice(k):
        return idx_all.at[pl.ds(pl.multiple_of(k * _C, _C), _C)]

    def out_slice(k):
        return out_hbm.at[pl.ds(pl.multiple_of(wbase + k * _C, _C), _C)]

    def fire_gather(k, buf):
        pltpu.async_copy(tab_hbm.at[idx_slice(k)], buf, gsem)

    def wait_gather(k, buf):
        pltpu.make_async_copy(tab_hbm.at[idx_slice(k)], buf, gsem).wait()

    def fire_write(k, buf):
        pltpu.async_copy(buf, out_slice(k), wsem)

    def wait_write(k, buf):
        pltpu.make_async_copy(buf, out_slice(k), wsem).wait()

    fire_gather(0, bufs[0])

    def outer(t, carry):
        for b in range(2):
            k = t * 2 + b
            cur = bufs[b]
            other = bufs[1 - b]

            @pl.when(k >= 1)
            def _():
                wait_write(k - 1, other)

            @pl.when(k + 1 < _CHUNKS)
            def _():
                fire_gather(k + 1, other)

            wait_gather(k, cur)
            fire_write(k, cur)
        return carry

    lax.fori_loop(0, _CHUNKS // 2, outer, 0)
    wait_write(_CHUNKS - 1, bufs[(_CHUNKS - 1) % 2])


@jax.jit
def _sc_gather(idx, tab):
    mesh = plsc.VectorSubcoreMesh(core_axis_name="c", subcore_axis_name="s")
    buf = pltpu.VMEM((_C, 2 * _D), jnp.float32)
    fn = pl.kernel(
        _sc_gather_body,
        out_type=jax.ShapeDtypeStruct((_NPAIR, 2 * _D), jnp.float32),
        mesh=mesh,
        scratch_types=[
            pltpu.VMEM((_PER_W,), jnp.int32),
            (buf, buf),
            pltpu.SemaphoreType.DMA,
            pltpu.SemaphoreType.DMA,
        ],
    )
    return fn(idx, tab)


_R = 4096
_BB = _R // _L


def _tc_body(u_ref, i_ref, w1t_ref, w1b_ref, b1_ref, gw_ref, mw_ref,
             sgt_ref, smt_ref, out_ref):
    prod = u_ref[:, :_D] * i_ref[:, :_D]
    gvec = jnp.dot(prod, gw_ref[...], preferred_element_type=jnp.float32)
    h = jnp.maximum(
        jnp.dot(u_ref[:, _D:], w1t_ref[...], preferred_element_type=jnp.float32)
        + jnp.dot(i_ref[:, _D:], w1b_ref[...], preferred_element_type=jnp.float32)
        + b1_ref[...], 0.0)
    mvec = jnp.dot(h, mw_ref[...], preferred_element_type=jnp.float32)
    out_ref[...] = (
        jnp.dot(sgt_ref[...], gvec, preferred_element_type=jnp.float32)
        + jnp.dot(smt_ref[...], mvec, preferred_element_type=jnp.float32))


@jax.jit
def _tc_dense(u_g, i_g, w1t, w1b, b1r, gw, mw, sgt, smt):
    n_blocks = _NPAIR // _R
    row_spec = pl.BlockSpec((_R, 2 * _D), lambda i: (i, 0))
    full = lambda shape: pl.BlockSpec(shape, lambda i: (0, 0))
    return pl.pallas_call(
        _tc_body,
        grid=(n_blocks,),
        in_specs=[
            row_spec, row_spec,
            full((_D, _D)), full((_D, _D)), full((1, _D)),
            full((_D, 1)), full((_D, 1)),
            full((_BB, _R)), full((_BB, _R)),
        ],
        out_specs=pl.BlockSpec((_BB, 1), lambda i: (i, 0)),
        out_shape=jax.ShapeDtypeStruct((_B, 1), jnp.float32),
    )(u_g, i_g, w1t, w1b, b1r, gw, mw, sgt, smt)


def kernel(user_id, item_id, mf_user_emb, mf_item_emb, gmf_w, gmf_b,
           mlp_user_emb, mlp_item_emb, mlp_w1, mlp_b1, mlp_w, mlp_b,
           fin_w, fin_b):
    uid = user_id.reshape(-1).astype(jnp.int32)
    iid = item_id.reshape(-1).astype(jnp.int32)

    eye64 = jnp.eye(_D, dtype=jnp.float32)
    item_cat = _sc_convert_pair(mf_item_emb.T, mlp_item_emb.T)
    user_cat = _convert_pair(mf_user_emb.T, mlp_user_emb.T, eye64)
    u_g = _sc_gather(uid, user_cat)
    i_g = _sc_gather(iid, item_cat)

    wg = fin_w[:_L, 0]
    wm = fin_w[_L:, 0]
    eye = jnp.eye(_BB, dtype=jnp.float32)
    sgt = jnp.kron(eye, wg[None, :])
    smt = jnp.kron(eye, wm[None, :])
    w1t = mlp_w1[:_D]
    w1b = mlp_w1[_D:]
    b1r = mlp_b1.reshape(1, _D)

    pred = _tc_dense(u_g, i_g, w1t, w1b, b1r, gmf_w, mlp_w, sgt, smt)
    cst = gmf_b[0] * jnp.sum(wg) + mlp_b[0] * jnp.sum(wm) + fin_b[0]
    return pred.reshape(_B) + cst


# speedup vs baseline: 3.1076x; 3.1076x over previous
import functools

import jax
import jax.numpy as jnp
from jax import lax
from jax.experimental import pallas as pl
from jax.experimental.pallas import tpu as pltpu
from jax.experimental.pallas import tpu_sc as plsc

_B = 4096
_L = 64
_D = 64
_V = 1000000
_NPAIR = _B * _L
_NW = 32
_PER_W = _NPAIR // _NW
_C = 128
_CHUNKS = _PER_W // _C
_VB = 16384


def _conv_body(pa_ref, pb_ref, eye_ref, out_ref):
    out_ref[:, :_D] = pa_ref[...].T
    out_ref[:, _D:] = pb_ref[...].T


@jax.jit
def _convert_pair(pa, pb, eye):
    grid = (pl.cdiv(_V, _VB),)
    in_spec = pl.BlockSpec((_D, _VB), lambda j: (0, j))
    return pl.pallas_call(
        _conv_body,
        grid=grid,
        in_specs=[in_spec, in_spec, pl.BlockSpec((_D, _D), lambda j: (0, 0))],
        out_specs=pl.BlockSpec((_VB, 2 * _D), lambda j: (j, 0)),
        out_shape=jax.ShapeDtypeStruct((_V, 2 * _D), jnp.float32),
    )(pa, pb, eye)


def _sc_gather_body(per_w, chunks, idx_hbm, tab_hbm, out_hbm,
                    idx_all, bufs, gsem, wsem):
    c = lax.axis_index("c")
    s = lax.axis_index("s")
    wid = s * 2 + c
    wbase = wid * per_w

    pltpu.sync_copy(idx_hbm.at[pl.ds(wbase, per_w)], idx_all)

    def idx_slice(k):
        return idx_all.at[pl.ds(pl.multiple_of(k * _C, _C), _C)]

    def out_slice(k):
        return out_hbm.at[pl.ds(pl.multiple_of(wbase + k * _C, _C), _C)]

    def fire_gather(k, buf):
        pltpu.async_copy(tab_hbm.at[idx_slice(k)], buf, gsem)

    def wait_gather(k, buf):
        pltpu.make_async_copy(tab_hbm.at[idx_slice(k)], buf, gsem).wait()

    def fire_write(k, buf):
        pltpu.async_copy(buf, out_slice(k), wsem)

    def wait_write(k, buf):
        pltpu.make_async_copy(buf, out_slice(k), wsem).wait()

    fire_gather(0, bufs[0])

    def outer(t, carry):
        for b in range(2):
            k = t * 2 + b
            cur = bufs[b]
            other = bufs[1 - b]

            @pl.when(k >= 1)
            def _():
                wait_write(k - 1, other)

            @pl.when(k + 1 < chunks)
            def _():
                fire_gather(k + 1, other)

            wait_gather(k, cur)
            fire_write(k, cur)
        return carry

    lax.fori_loop(0, chunks // 2, outer, 0)
    wait_write(chunks - 1, bufs[(chunks - 1) % 2])


@functools.partial(jax.jit, static_argnums=2)
def _sc_gather(idx, tab, npair):
    per_w = npair // _NW
    chunks = per_w // _C
    mesh = plsc.VectorSubcoreMesh(core_axis_name="c", subcore_axis_name="s")
    buf = pltpu.VMEM((_C, 2 * _D), jnp.float32)
    fn = pl.kernel(
        functools.partial(_sc_gather_body, per_w, chunks),
        out_type=jax.ShapeDtypeStruct((npair, 2 * _D), jnp.float32),
        mesh=mesh,
        scratch_types=[
            pltpu.VMEM((per_w,), jnp.int32),
            (buf, buf),
            pltpu.SemaphoreType.DMA,
            pltpu.SemaphoreType.DMA,
        ],
    )
    return fn(idx, tab)


_R = 4096
_BB = _R // _L


def _tc_body(u_ref, i_ref, w1t_ref, w1b_ref, b1_ref, gw_ref, mw_ref,
             sgt_ref, smt_ref, out_ref):
    prod = u_ref[:, :_D] * i_ref[:, :_D]
    gvec = jnp.dot(prod, gw_ref[...], preferred_element_type=jnp.float32)
    h = jnp.maximum(
        jnp.dot(u_ref[:, _D:], w1t_ref[...], preferred_element_type=jnp.float32)
        + jnp.dot(i_ref[:, _D:], w1b_ref[...], preferred_element_type=jnp.float32)
        + b1_ref[...], 0.0)
    mvec = jnp.dot(h, mw_ref[...], preferred_element_type=jnp.float32)
    out_ref[...] = (
        jnp.dot(sgt_ref[...], gvec, preferred_element_type=jnp.float32)
        + jnp.dot(smt_ref[...], mvec, preferred_element_type=jnp.float32))


@functools.partial(jax.jit, static_argnums=(9, 10))
def _tc_dense(u_g, i_g, w1t, w1b, b1r, gw, mw, sgt, smt, u_off, n_blocks):
    u_spec = pl.BlockSpec((_R, 2 * _D), lambda i: (i + u_off, 0))
    i_spec = pl.BlockSpec((_R, 2 * _D), lambda i: (i, 0))
    full = lambda shape: pl.BlockSpec(shape, lambda i: (0, 0))
    return pl.pallas_call(
        _tc_body,
        grid=(n_blocks,),
        in_specs=[
            u_spec, i_spec,
            full((_D, _D)), full((_D, _D)), full((1, _D)),
            full((_D, 1)), full((_D, 1)),
            full((_BB, _R)), full((_BB, _R)),
        ],
        out_specs=pl.BlockSpec((_BB, 1), lambda i: (i, 0)),
        out_shape=jax.ShapeDtypeStruct((n_blocks * _BB, 1), jnp.float32),
    )(u_g, i_g, w1t, w1b, b1r, gw, mw, sgt, smt)


def kernel(user_id, item_id, mf_user_emb, mf_item_emb, gmf_w, gmf_b,
           mlp_user_emb, mlp_item_emb, mlp_w1, mlp_b1, mlp_w, mlp_b,
           fin_w, fin_b):
    uid = user_id.reshape(-1).astype(jnp.int32)
    iid = item_id.reshape(-1).astype(jnp.int32)

    eye64 = jnp.eye(_D, dtype=jnp.float32)
    user_cat = _convert_pair(mf_user_emb.T, mlp_user_emb.T, eye64)
    u_g = _sc_gather(uid, user_cat, _NPAIR)
    item_cat = _convert_pair(mf_item_emb.T, mlp_item_emb.T, eye64)
    half = _NPAIR // 2
    i_g_a = _sc_gather(iid[:half], item_cat, half)
    i_g_b = _sc_gather(iid[half:], item_cat, half)

    wg = fin_w[:_L, 0]
    wm = fin_w[_L:, 0]
    eye = jnp.eye(_BB, dtype=jnp.float32)
    sgt = jnp.kron(eye, wg[None, :])
    smt = jnp.kron(eye, wm[None, :])
    w1t = mlp_w1[:_D]
    w1b = mlp_w1[_D:]
    b1r = mlp_b1.reshape(1, _D)

    hb = half // _R
    pred_a = _tc_dense(u_g, i_g_a, w1t, w1b, b1r, gmf_w, mlp_w, sgt, smt,
                       0, hb)
    pred_b = _tc_dense(u_g, i_g_b, w1t, w1b, b1r, gmf_w, mlp_w, sgt, smt,
                       hb, hb)
    cst = gmf_b[0] * jnp.sum(wg) + mlp_b[0] * jnp.sum(wm) + fin_b[0]
    return jnp.concatenate([pred_a, pred_b], axis=0).reshape(_B) + cst
